# baseline (speedup 1.0000x reference)
import jax
import jax.numpy as jnp
from jax.experimental import pallas as pl
from jax.experimental.pallas import tpu as pltpu

_E = 320000
_EMB = 128
_OUT = 256
_NR = 6
_N_SEG = 10000.0
_BE = 16000
_NCORES = 2
_STEPS = _E // (_BE * _NCORES)
_NR8 = 8


def _stream_kernel(rbfT_ref, m_ref, part_ref, acc_ref):
    c = pl.program_id(0)
    j = pl.program_id(1)

    @pl.when(j == 0)
    def _init():
        acc_ref[...] = jnp.zeros_like(acc_ref)

    e0 = (c * _STEPS + j) * _BE
    acc_ref[...] += jnp.dot(rbfT_ref[:, pl.ds(e0, _BE)], m_ref[...],
                            preferred_element_type=jnp.float32)

    @pl.when(j == _STEPS - 1)
    def _flush():
        part_ref[...] = acc_ref[...]


def _tail_kernel(part_ref, wrbfT_ref, wup_ref, wd0_ref, wd1_ref, wd2_ref,
                 bias_ref, out_ref):
    acc = part_ref[0:_NR8, :] + part_ref[_NR8:2 * _NR8, :]
    s = jnp.sum(wrbfT_ref[...] * acc, axis=0, keepdims=True)
    dn = (((1,), (1,)), ((), ()))
    v = jax.lax.dot_general(s, wup_ref[...], dn,
                            preferred_element_type=jnp.float32)
    v = jax.lax.dot_general(v, wd0_ref[...], dn,
                            preferred_element_type=jnp.float32)
    v += _N_SEG * bias_ref[0:1, :]
    v = jax.lax.dot_general(v, wd1_ref[...], dn,
                            preferred_element_type=jnp.float32)
    v += _N_SEG * bias_ref[1:2, :]
    v = jax.lax.dot_general(v, wd2_ref[...], dn,
                            preferred_element_type=jnp.float32)
    v += _N_SEG * bias_ref[2:3, :]
    out_ref[...] = v


def kernel(m, rbf, edge_index, W_rbf, W_up, W_d0, b_d0, W_d1, b_d1, W_d2, b_d2):
    del edge_index
    rbfT = jnp.pad(rbf.T, ((0, _NR8 - _NR), (0, 0)))
    wrbfT = jnp.pad(W_rbf.T, ((0, _NR8 - _NR), (0, 0)))
    bias = jnp.stack([b_d0, b_d1, b_d2])
    part = pl.pallas_call(
        _stream_kernel,
        grid=(_NCORES, _STEPS),
        in_specs=[
            pl.BlockSpec((_NR8, _E), lambda c, j: (0, 0)),
            pl.BlockSpec((_BE, _EMB), lambda c, j: (c * _STEPS + j, 0)),
        ],
        out_specs=pl.BlockSpec((_NR8, _EMB), lambda c, j: (c, 0)),
        out_shape=jax.ShapeDtypeStruct((_NCORES * _NR8, _EMB), jnp.float32),
        scratch_shapes=[pltpu.VMEM((_NR8, _EMB), jnp.float32)],
        compiler_params=pltpu.CompilerParams(
            dimension_semantics=("parallel", "arbitrary")),
    )(rbfT, m)
    return pl.pallas_call(
        _tail_kernel,
        out_shape=jax.ShapeDtypeStruct((1, _OUT), jnp.float32),
    )(part, wrbfT, W_up, W_d0, W_d1, W_d2, bias)


# speedup vs baseline: 1.2249x; 1.2249x over previous
import jax
import jax.numpy as jnp
from jax.experimental import pallas as pl
from jax.experimental.pallas import tpu as pltpu

_E = 320000
_EMB = 128
_OUT = 256
_NR = 6
_N_SEG = 10000.0
_BE = 16000


def _ppblock_kernel(rbfT_ref, m_ref, wrbfT_ref, wup_ref, wd0_ref, wd1_ref,
                    wd2_ref, bias_ref, out_ref, acc_ref):
    i = pl.program_id(0)

    @pl.when(i == 0)
    def _init():
        acc_ref[...] = jnp.zeros_like(acc_ref)

    acc_ref[...] += jnp.dot(rbfT_ref[:, pl.ds(i * _BE, _BE)], m_ref[...],
                            preferred_element_type=jnp.float32)

    @pl.when(i == pl.num_programs(0) - 1)
    def _finish():
        s = jnp.sum(wrbfT_ref[...] * acc_ref[...], axis=0, keepdims=True)
        dn = (((1,), (1,)), ((), ()))
        v = jax.lax.dot_general(s, wup_ref[...], dn,
                                preferred_element_type=jnp.float32)
        v = jax.lax.dot_general(v, wd0_ref[...], dn,
                                preferred_element_type=jnp.float32)
        v += _N_SEG * bias_ref[0:1, :]
        v = jax.lax.dot_general(v, wd1_ref[...], dn,
                                preferred_element_type=jnp.float32)
        v += _N_SEG * bias_ref[1:2, :]
        v = jax.lax.dot_general(v, wd2_ref[...], dn,
                                preferred_element_type=jnp.float32)
        v += _N_SEG * bias_ref[2:3, :]
        out_ref[...] = v


def kernel(m, rbf, edge_index, W_rbf, W_up, W_d0, b_d0, W_d1, b_d1, W_d2, b_d2):
    del edge_index
    rbfT = rbf.T
    wrbfT = W_rbf.T
    bias = jnp.stack([b_d0, b_d1, b_d2])
    grid = _E // _BE
    return pl.pallas_call(
        _ppblock_kernel,
        grid=(grid,),
        in_specs=[
            pl.BlockSpec((_NR, _E), lambda i: (0, 0)),
            pl.BlockSpec((_BE, _EMB), lambda i: (i, 0)),
            pl.BlockSpec((_NR, _EMB), lambda i: (0, 0)),
            pl.BlockSpec((_OUT, _EMB), lambda i: (0, 0)),
            pl.BlockSpec((_OUT, _OUT), lambda i: (0, 0)),
            pl.BlockSpec((_OUT, _OUT), lambda i: (0, 0)),
            pl.BlockSpec((_OUT, _OUT), lambda i: (0, 0)),
            pl.BlockSpec((3, _OUT), lambda i: (0, 0)),
        ],
        out_specs=pl.BlockSpec((1, _OUT), lambda i: (0, 0)),
        out_shape=jax.ShapeDtypeStruct((1, _OUT), jnp.float32),
        scratch_shapes=[pltpu.VMEM((_NR, _EMB), jnp.float32)],
        compiler_params=pltpu.CompilerParams(
            allow_input_fusion=[True, False, False, False, False, False,
                                False, False]),
    )(rbfT, m, wrbfT, W_up, W_d0, W_d1, W_d2, bias)
